# baseline (speedup 1.0000x reference)
import functools

import jax
import jax.numpy as jnp
from jax import lax
from jax.experimental import pallas as pl
from jax.experimental.pallas import tpu as pltpu
from jax.experimental.pallas import tpu_sc as plsc

_L = 16
_SL = 8
_LN = 128


def _make_kwinners(n_rt, n_ct, RT):
  assert n_rt % (32 * 2 * RT) == 0
  rt_w = n_rt // 32
  n_chunks = rt_w // RT
  n_k = n_ct * _LN // _L
  pix = RT * _SL
  mesh = plsc.VectorSubcoreMesh(core_axis_name="c", subcore_axis_name="s")

  @functools.partial(
      pl.kernel,
      mesh=mesh,
      out_type=jax.ShapeDtypeStruct((n_rt, n_ct, _SL, _LN), jnp.float32),
      compiler_params=pltpu.CompilerParams(
          needs_layout_passes=False, use_tc_tiling_on_sc=True),
      scratch_types=[
          pltpu.VMEM((RT, n_ct, _SL, _LN), jnp.float32),
          pltpu.VMEM((RT, n_ct, _SL, _LN), jnp.float32),
          pltpu.VMEM((RT, n_ct, _SL, _LN), jnp.float32),
          pltpu.VMEM((RT, n_ct, _SL, _LN), jnp.float32),
          pltpu.SemaphoreType.DMA,
          pltpu.SemaphoreType.DMA,
          pltpu.SemaphoreType.DMA,
          pltpu.SemaphoreType.DMA,
      ],
  )
  def kw(x_hbm, o_hbm, ibuf_a, ibuf_b, obuf_a, obuf_b,
         rsem_a, rsem_b, wsem_a, wsem_b):
    wid = lax.axis_index("s") * 2 + lax.axis_index("c")
    base_rt = wid * rt_w
    lane = lax.iota(jnp.int32, _L)
    big_i = jnp.full((_L,), n_ct * _LN, jnp.int32)
    ibufs = (ibuf_a, ibuf_b)
    obufs = (obuf_a, obuf_b)
    rsems = (rsem_a, rsem_b)
    wsems = (wsem_a, wsem_b)

    def rd(ci):
      return x_hbm.at[pl.ds(base_rt + ci * RT, RT), :, :, :]

    def wr(ci):
      return o_hbm.at[pl.ds(base_rt + ci * RT, RT), :, :, :]

    pltpu.async_copy(rd(0), ibuf_a, rsem_a)
    pltpu.async_copy(rd(1), ibuf_b, rsem_b)

    def chunk_body(g, _):
      for p in range(2):
        ci = 2 * g + p
        pltpu.make_async_copy(rd(ci), ibufs[p], rsems[p]).wait()

        @pl.when(g > 0)
        def _():
          pltpu.make_async_copy(obufs[p], wr(ci - 2), wsems[p]).wait()

        ib = ibufs[p]
        ob = obufs[p]

        def pix_body(q, _):
          t = q >> 3
          rs = q & 7
          m = ib[t, 0, rs, pl.ds(0, _L)]
          i = lane
          for kk in range(1, n_k):
            v = ib[t, kk >> 3, rs, pl.ds((kk & 7) * _L, _L)]
            gt = v > m
            m = jnp.where(gt, v, m)
            i = jnp.where(gt, kk * _L + lane, i)
          mx = jnp.max(m)
          wi = jnp.min(jnp.where(m == mx, i, big_i))
          for kk in range(n_k):
            cvec = kk * _L + lane
            ob[t, kk >> 3, rs, pl.ds((kk & 7) * _L, _L)] = jnp.where(
                cvec == wi, mx, jnp.float32(0))
          return 0

        lax.fori_loop(0, pix, pix_body, 0)
        pltpu.async_copy(ob, wr(ci), wsems[p])

        @pl.when(ci + 2 < n_chunks)
        def _():
          pltpu.async_copy(rd(ci + 2), ibufs[p], rsems[p])

      return 0

    lax.fori_loop(0, n_chunks // 2, chunk_body, 0)

    for p in range(2):
      pltpu.make_async_copy(obufs[p], wr(n_chunks - 2 + p), wsems[p]).wait()

  return kw


def kernel(x, k):
  B, C, H, W = x.shape
  n_rt = B * H * W // _SL
  n_ct = C // _LN
  xt = jnp.transpose(x, (0, 2, 3, 1))
  x4 = jnp.transpose(
      xt.reshape(n_rt, _SL, n_ct, _LN), (0, 2, 1, 3))
  o4 = _make_kwinners(n_rt, n_ct, 4)(x4)
  ot = jnp.transpose(o4, (0, 2, 1, 3)).reshape(B, H, W, C)
  return jnp.transpose(ot, (0, 3, 1, 2))
